# Initial kernel scaffold; baseline (speedup 1.0000x reference)
import jax
import jax.numpy as jnp
from jax.experimental import pallas as pl


def kernel(pred, target):
    raise NotImplementedError("write your pallas kernel here")


# speedup vs baseline: 2.1328x; 2.1328x over previous
import jax
import jax.numpy as jnp
from jax.experimental import pallas as pl
from jax.experimental.pallas import tpu as pltpu

_V = 100000
_EPS = 0.1
_CONF = 1.0 - _EPS
_SMOOTH = _EPS / (_V - 2)
_IGNORE = 0

_BV = 2048
_NV = (_V + _BV - 1) // _BV


def _loss_kernel(x_ref, t_ref, out_ref, m_ref, s_ref, sum_ref, pt_ref, p0_ref):
    j = pl.program_id(0)
    x = x_ref[...]
    t = t_ref[...]
    r = x.shape[0]

    col_ids = j * _BV + jax.lax.broadcasted_iota(jnp.int32, (r, _BV), 1)
    valid = col_ids < _V
    xm = jnp.where(valid, x, -jnp.inf)
    blk_max = jnp.max(xm, axis=1, keepdims=True)
    blk_sum = jnp.sum(jnp.where(valid, x, 0.0), axis=1, keepdims=True)
    pt_blk = jnp.sum(jnp.where(col_ids == t, x, 0.0), axis=1, keepdims=True)

    @pl.when(j == 0)
    def _():
        m_ref[...] = blk_max
        s_ref[...] = jnp.sum(jnp.exp(xm - blk_max), axis=1, keepdims=True)
        sum_ref[...] = blk_sum
        pt_ref[...] = pt_blk
        p0_ref[...] = x[:, 0:1]

    @pl.when(j > 0)
    def _():
        m_old = m_ref[...]
        m_new = jnp.maximum(m_old, blk_max)
        s_ref[...] = s_ref[...] * jnp.exp(m_old - m_new) + jnp.sum(
            jnp.exp(xm - m_new), axis=1, keepdims=True)
        m_ref[...] = m_new
        sum_ref[...] = sum_ref[...] + blk_sum
        pt_ref[...] = pt_ref[...] + pt_blk

    @pl.when(j == _NV - 1)
    def _():
        lse = m_ref[...] + jnp.log(s_ref[...])
        logp_t = pt_ref[...] - lse
        logp_0 = p0_ref[...] - lse
        s_row = sum_ref[...] - jnp.float32(_V) * lse
        contrib = _SMOOTH * (s_row - logp_t - logp_0) + _CONF * logp_t
        rmask = t != _IGNORE
        contrib = jnp.where(rmask, contrib, 0.0)
        n_valid = jnp.sum(rmask.astype(jnp.float32))
        loss = -jnp.sum(contrib) / jnp.maximum(n_valid, 1.0)
        out_ref[...] = loss.reshape(1, 1)


def kernel(pred, target):
    pred2 = pred.reshape(-1, pred.shape[-1])
    n = pred2.shape[0]
    t2 = target.reshape(n, 1)
    out = pl.pallas_call(
        _loss_kernel,
        grid=(_NV,),
        in_specs=[
            pl.BlockSpec((n, _BV), lambda j: (0, j)),
            pl.BlockSpec((n, 1), lambda j: (0, 0)),
        ],
        out_specs=pl.BlockSpec((1, 1), lambda j: (0, 0)),
        out_shape=jax.ShapeDtypeStruct((1, 1), jnp.float32),
        scratch_shapes=[pltpu.VMEM((n, 1), jnp.float32) for _ in range(5)],
    )(pred2, t2)
    return out[0, 0]
